# Initial kernel scaffold; baseline (speedup 1.0000x reference)
import jax
import jax.numpy as jnp
from jax.experimental import pallas as pl


def kernel(embedding, centers, logits):
    raise NotImplementedError("write your pallas kernel here")


# speedup vs baseline: 2.7515x; 2.7515x over previous
import functools

import jax
import jax.numpy as jnp
from jax import lax
from jax.experimental import pallas as pl
from jax.experimental.pallas import tpu as pltpu
from jax.experimental.pallas import tpu_sc as plsc

_N = 100000
_D = 128
_C = 64

_BN = 2048

_SC_B = 800
_SC_NBLK = _N // _SC_B
_SC_NW = 32
_SC_ITERS = -(-_SC_NBLK // _SC_NW)


def _argmax_body(logits_ref, assign_ref, counts_ref):
    i = pl.program_id(0)
    blk = logits_ref[...]
    col = lax.broadcasted_iota(jnp.int32, blk.shape, 1)
    m = jnp.max(blk, axis=-1, keepdims=True)
    amax = jnp.min(jnp.where(blk == m, col, _C), axis=-1).astype(jnp.int32)
    assign_ref[...] = amax

    @pl.when(i == 0)
    def _():
        counts_ref[...] = jnp.zeros_like(counts_ref)

    rows = i * _BN + lax.broadcasted_iota(jnp.int32, (_BN, 1), 0)
    valid = (rows < _N).astype(jnp.float32)
    onehot = (amax[:, None] == col).astype(jnp.float32) * valid
    counts_ref[...] += jnp.sum(onehot, axis=0)


def _argmax_call(logits):
    grid = -(-_N // _BN)
    return pl.pallas_call(
        _argmax_body,
        grid=(grid,),
        in_specs=[pl.BlockSpec((_BN, _C), lambda i: (i, 0))],
        out_specs=[
            pl.BlockSpec((_BN,), lambda i: (i,)),
            pl.BlockSpec((_C,), lambda i: (0,)),
        ],
        out_shape=[
            jax.ShapeDtypeStruct((grid * _BN,), jnp.int32),
            jax.ShapeDtypeStruct((_C,), jnp.float32),
        ],
    )(logits)


def _segsum_body(emb_hbm, assign_hbm, zeros_hbm, out_hbm, emb_buf, idx_buf, accum):
    cid = lax.axis_index("c")
    sid = lax.axis_index("s")
    wid = sid * 2 + cid

    @pl.when(sid == 0)
    def _():
        pltpu.sync_copy(zeros_hbm, accum)

    plsc.subcore_barrier()

    for t in range(_SC_ITERS):
        b = t * _SC_NW + wid

        @pl.when(b < _SC_NBLK)
        def _():
            base = b * _SC_B
            pltpu.sync_copy(emb_hbm.at[pl.ds(base, _SC_B), :], emb_buf)
            pltpu.sync_copy(assign_hbm.at[pl.ds(base, _SC_B)], idx_buf)
            pltpu.sync_copy(emb_buf, accum.at[idx_buf], add=True)

    plsc.subcore_barrier()

    @pl.when(sid == 0)
    def _():
        pltpu.sync_copy(accum, out_hbm.at[cid])


def _segsum_call(embedding, assign):
    mesh = plsc.VectorSubcoreMesh(
        core_axis_name="c", subcore_axis_name="s", num_cores=2, num_subcores=16
    )
    zeros = jnp.zeros((_C, _D), jnp.float32)
    f = pl.kernel(
        _segsum_body,
        out_type=jax.ShapeDtypeStruct((2, _C, _D), jnp.float32),
        mesh=mesh,
        scratch_types=[
            pltpu.VMEM((_SC_B, _D), jnp.float32),
            pltpu.VMEM((_SC_B,), jnp.int32),
            pltpu.VMEM_SHARED((_C, _D), jnp.float32),
        ],
    )
    return f(embedding, assign, zeros)


def _finalize_body(sums_ref, counts_ref, centers_ref, out_ref):
    sums = sums_ref[0] + sums_ref[1]
    counts = counts_ref[...]
    centroids = sums / jnp.maximum(counts, 1.0)[:, None]
    delta = centers_ref[...] - centroids
    dist = jnp.sqrt(jnp.sum(delta * delta, axis=-1))
    out_ref[...] = jnp.where(counts > 0, dist, 0.0)


def _finalize_call(sums_partial, counts, centers):
    return pl.pallas_call(
        _finalize_body,
        out_shape=jax.ShapeDtypeStruct((_C,), jnp.float32),
    )(sums_partial, counts, centers)


def kernel(embedding, centers, logits):
    assign, counts = _argmax_call(logits)
    sums_partial = _segsum_call(embedding, assign)
    return _finalize_call(sums_partial, counts, centers)
